# Initial kernel scaffold; baseline (speedup 1.0000x reference)
import jax
import jax.numpy as jnp
from jax.experimental import pallas as pl


def kernel(x, mask):
    raise NotImplementedError("write your pallas kernel here")


# speedup vs baseline: 1.2278x; 1.2278x over previous
import jax
import jax.numpy as jnp
from jax.experimental import pallas as pl


def _copy2_kernel(x_ref, a_ref, b_ref):
    v = x_ref[...]
    a_ref[...] = v
    b_ref[...] = v


def kernel(x, mask):
    B, S, D = x.shape
    blk = (1, S, D)
    spec = pl.BlockSpec(blk, lambda i: (i, 0, 0))
    out = pl.pallas_call(
        _copy2_kernel,
        grid=(B,),
        in_specs=[spec],
        out_specs=[spec, spec],
        out_shape=[
            jax.ShapeDtypeStruct(x.shape, x.dtype),
            jax.ShapeDtypeStruct(x.shape, x.dtype),
        ],
    )(x)
    return (out[0], out[1])
